# baseline (speedup 1.0000x reference)
import functools

import jax
import jax.numpy as jnp
from jax import lax
from jax.experimental import pallas as pl
from jax.experimental.pallas import tpu as pltpu
from jax.experimental.pallas import tpu_sc as plsc

_MU = 3.5
_CHUNK = 32


@functools.lru_cache(maxsize=None)
def _build_sc_kernel(B, K):
    info = plsc.get_sparse_core_info()
    NC, NS, L = info.num_cores, info.num_subcores, info.num_lanes
    NW = NC * NS
    assert B % (8 * NW) == 0 and K % L == 0
    b_per_w = B // NW
    n_chunks = b_per_w // _CHUNK
    mesh = plsc.VectorSubcoreMesh(core_axis_name="c", subcore_axis_name="s",
                                  num_cores=NC, num_subcores=NS)

    @functools.partial(
        pl.kernel,
        out_type=jax.ShapeDtypeStruct((B,), jnp.float32),
        mesh=mesh,
        scratch_types=[
            pltpu.VMEM((b_per_w,), jnp.int32),
            pltpu.VMEM((b_per_w,), jnp.int32),
            pltpu.VMEM((b_per_w,), jnp.int32),
            pltpu.VMEM((b_per_w,), jnp.int32),
            pltpu.VMEM((b_per_w,), jnp.int32),
            pltpu.VMEM((b_per_w,), jnp.int32),
            pltpu.VMEM((_CHUNK * 8, K), jnp.float32),
            pltpu.VMEM((_CHUNK * 8, K), jnp.float32),
            pltpu.VMEM((b_per_w,), jnp.float32),
            pltpu.VMEM((b_per_w,), jnp.float32),
            pltpu.VMEM((b_per_w,), jnp.float32),
            pltpu.SemaphoreType.DMA,
            pltpu.SemaphoreType.DMA,
            pltpu.SemaphoreType.DMA,
            pltpu.SemaphoreType.DMA,
        ],
        compiler_params=pltpu.CompilerParams(needs_layout_passes=False),
        interpret=False,
    )
    def sc_kernel(uidx_hbm, iidx_hbm, p_hbm, q_hbm, bu_hbm, bi_hbm, out_hbm,
                  idx_u, idx_i, blk_u, blk_i, sub_u, sub_i,
                  blocks_p, blocks_q, bu_v, bi_v, out_v,
                  sem0, sem1, sem2, sem3):
        wid = lax.axis_index("s") * NC + lax.axis_index("c")
        base = wid * b_per_w
        for c in range(b_per_w // 128):
            pltpu.sync_copy(uidx_hbm.at[pl.ds(base + c * 128, 128)],
                            idx_u.at[pl.ds(c * 128, 128)])
            pltpu.sync_copy(iidx_hbm.at[pl.ds(base + c * 128, 128)],
                            idx_i.at[pl.ds(c * 128, 128)])

        bias_copies = []
        for c in range(b_per_w // 128):
            sl = pl.ds(c * 128, 128)
            bias_copies.append(
                pltpu.async_copy(bu_hbm.at[idx_u.at[sl]], bu_v.at[sl], sem2))
            bias_copies.append(
                pltpu.async_copy(bi_hbm.at[idx_i.at[sl]], bi_v.at[sl], sem3))

        def split(g, carry):
            s = pl.ds(g * L, L)
            u = idx_u[s]
            i = idx_i[s]
            blk_u[s] = (u >> 3) * 8
            sub_u[s] = u & 7
            blk_i[s] = (i >> 3) * 8
            sub_i[s] = i & 7
            return carry

        lax.fori_loop(0, b_per_w // L, split, 0)

        lane = lax.iota(jnp.int32, L)

        def chunk_body(c, carry):
            cbase = c * _CHUNK

            for g in range(_CHUNK // L):
                ub_vec = blk_u[pl.ds(cbase + g * L, L)]
                ib_vec = blk_i[pl.ds(cbase + g * L, L)]
                for j in range(L):
                    pltpu.async_copy(
                        p_hbm.at[pl.ds(pl.multiple_of(ub_vec[j], 8), 8)],
                        blocks_p.at[pl.ds((g * L + j) * 8, 8)], sem0)
                    pltpu.async_copy(
                        q_hbm.at[pl.ds(pl.multiple_of(ib_vec[j], 8), 8)],
                        blocks_q.at[pl.ds((g * L + j) * 8, 8)], sem1)
            pltpu.make_async_copy(
                p_hbm.at[pl.ds(0, _CHUNK * 8)], blocks_p, sem0).wait()
            pltpu.make_async_copy(
                q_hbm.at[pl.ds(0, _CHUNK * 8)], blocks_q, sem1).wait()
            for g in range(_CHUNK // L):
                s = pl.ds(cbase + g * L, L)
                slotp = (g * L + lane) * 8 + sub_u[s]
                slotq = (g * L + lane) * 8 + sub_i[s]
                acc = jnp.zeros((L,), jnp.float32)
                for j in range(K):
                    col = jnp.full((L,), j, jnp.int32)
                    pv = plsc.load_gather(blocks_p, [slotp, col])
                    qv = plsc.load_gather(blocks_q, [slotq, col])
                    acc = acc + pv * qv
                out_v[s] = acc
            return carry

        lax.fori_loop(0, n_chunks, chunk_body, 0)
        for cp in bias_copies:
            cp.wait()

        def finish(g, carry):
            s = pl.ds(g * L, L)
            out_v[s] = out_v[s] + (bu_v[s] + bi_v[s] + _MU)
            return carry

        lax.fori_loop(0, b_per_w // L, finish, 0)
        pltpu.sync_copy(out_v, out_hbm.at[pl.ds(base, b_per_w)])

    return sc_kernel


def kernel(user_idx, item_idx, P, Q, b_u, b_i):
    B = user_idx.shape[0]
    K = P.shape[1]
    sc_kernel = _build_sc_kernel(B, K)
    return sc_kernel(user_idx.astype(jnp.int32), item_idx.astype(jnp.int32),
                     P, Q, b_u.reshape(-1), b_i.reshape(-1))


# speedup vs baseline: 1.5010x; 1.5010x over previous
import functools

import jax
import jax.numpy as jnp
from jax import lax
from jax.experimental import pallas as pl
from jax.experimental.pallas import tpu as pltpu
from jax.experimental.pallas import tpu_sc as plsc

_MU = 3.5
_CHUNK = 32


@functools.lru_cache(maxsize=None)
def _build_sc_kernel(B, K):
    info = plsc.get_sparse_core_info()
    NC, NS, L = info.num_cores, info.num_subcores, info.num_lanes
    NW = NC * NS
    assert B % (8 * NW) == 0 and K % L == 0
    b_per_w = B // NW
    n_chunks = b_per_w // _CHUNK
    mesh = plsc.VectorSubcoreMesh(core_axis_name="c", subcore_axis_name="s",
                                  num_cores=NC, num_subcores=NS)

    @functools.partial(
        pl.kernel,
        out_type=jax.ShapeDtypeStruct((B,), jnp.float32),
        mesh=mesh,
        scratch_types=[
            pltpu.VMEM((b_per_w,), jnp.int32),
            pltpu.VMEM((b_per_w,), jnp.int32),
            pltpu.VMEM((b_per_w,), jnp.int32),
            pltpu.VMEM((b_per_w,), jnp.int32),
            pltpu.VMEM((b_per_w,), jnp.int32),
            pltpu.VMEM((b_per_w,), jnp.int32),
            pltpu.VMEM((_CHUNK, 8, K), jnp.float32),
            pltpu.VMEM((_CHUNK, 8, K), jnp.float32),
            pltpu.VMEM((b_per_w,), jnp.float32),
            pltpu.VMEM((b_per_w,), jnp.float32),
            pltpu.VMEM((b_per_w,), jnp.float32),
            pltpu.SemaphoreType.DMA,
            pltpu.SemaphoreType.DMA,
            pltpu.SemaphoreType.DMA,
            pltpu.SemaphoreType.DMA,
        ],
        compiler_params=pltpu.CompilerParams(needs_layout_passes=False),
        interpret=False,
    )
    def sc_kernel(uidx_hbm, iidx_hbm, p_hbm, q_hbm, bu_hbm, bi_hbm, out_hbm,
                  idx_u, idx_i, blk_u, blk_i, sub_u, sub_i,
                  blocks_p, blocks_q, bu_v, bi_v, out_v,
                  sem0, sem1, sem2, sem3):
        wid = lax.axis_index("s") * NC + lax.axis_index("c")
        base = wid * b_per_w
        for c in range(b_per_w // 128):
            pltpu.sync_copy(uidx_hbm.at[pl.ds(base + c * 128, 128)],
                            idx_u.at[pl.ds(c * 128, 128)])
            pltpu.sync_copy(iidx_hbm.at[pl.ds(base + c * 128, 128)],
                            idx_i.at[pl.ds(c * 128, 128)])

        bias_copies = []
        for c in range(b_per_w // 128):
            sl = pl.ds(c * 128, 128)
            bias_copies.append(
                pltpu.async_copy(bu_hbm.at[idx_u.at[sl]], bu_v.at[sl], sem2))
            bias_copies.append(
                pltpu.async_copy(bi_hbm.at[idx_i.at[sl]], bi_v.at[sl], sem3))

        def split(g, carry):
            s = pl.ds(g * L, L)
            u = idx_u[s]
            i = idx_i[s]
            blk_u[s] = u >> 3
            sub_u[s] = u & 7
            blk_i[s] = i >> 3
            sub_i[s] = i & 7
            return carry

        lax.fori_loop(0, b_per_w // L, split, 0)

        lane = lax.iota(jnp.int32, L)

        def chunk_body(c, carry):
            cbase = c * _CHUNK

            for g in range(_CHUNK // L):
                ub_vec = blk_u[pl.ds(cbase + g * L, L)]
                ib_vec = blk_i[pl.ds(cbase + g * L, L)]
                for j in range(L):
                    pltpu.async_copy(
                        p_hbm.at[ub_vec[j]], blocks_p.at[g * L + j], sem0)
                    pltpu.async_copy(
                        q_hbm.at[ib_vec[j]], blocks_q.at[g * L + j], sem1)
            pltpu.make_async_copy(
                p_hbm.at[pl.ds(0, _CHUNK)], blocks_p, sem0).wait()
            pltpu.make_async_copy(
                q_hbm.at[pl.ds(0, _CHUNK)], blocks_q, sem1).wait()
            for g in range(_CHUNK // L):
                s = pl.ds(cbase + g * L, L)
                slot = g * L + lane
                su = sub_u[s]
                si = sub_i[s]
                acc = jnp.zeros((L,), jnp.float32)
                for j in range(K):
                    col = jnp.full((L,), j, jnp.int32)
                    pv = plsc.load_gather(blocks_p, [slot, su, col])
                    qv = plsc.load_gather(blocks_q, [slot, si, col])
                    acc = acc + pv * qv
                out_v[s] = acc
            return carry

        lax.fori_loop(0, n_chunks, chunk_body, 0)
        for cp in bias_copies:
            cp.wait()

        def finish(g, carry):
            s = pl.ds(g * L, L)
            out_v[s] = out_v[s] + (bu_v[s] + bi_v[s] + _MU)
            return carry

        lax.fori_loop(0, b_per_w // L, finish, 0)
        pltpu.sync_copy(out_v, out_hbm.at[pl.ds(base, b_per_w)])

    return sc_kernel


def kernel(user_idx, item_idx, P, Q, b_u, b_i):
    B = user_idx.shape[0]
    K = P.shape[1]
    sc_kernel = _build_sc_kernel(B, K)
    p3 = P.reshape(P.shape[0] // 8, 8, K)
    q3 = Q.reshape(Q.shape[0] // 8, 8, K)
    return sc_kernel(user_idx.astype(jnp.int32), item_idx.astype(jnp.int32),
                     p3, q3, b_u.reshape(-1), b_i.reshape(-1))
